# baseline (speedup 1.0000x reference)
import functools

import jax
import jax.numpy as jnp
from jax import lax
from jax.experimental import pallas as pl
from jax.experimental.pallas import tpu as pltpu
from jax.experimental.pallas import tpu_sc as plsc

_N = 10000
_E = 320000
_D = 128
_H = 10
_L = 16
_NC = 2
_NS = 16
_NW = _NC * _NS
_NPAD = 10240
_RPT = _NPAD // _NS
_EPW = _E // _NW
_CHUNK = 80
_NCHUNKS = _EPW // _CHUNK


def _project_n_body(feat_ref, w_ref, out_ref):
    p = jnp.dot(feat_ref[...], w_ref[...], preferred_element_type=jnp.float32)
    lane = lax.broadcasted_iota(jnp.int32, p.shape, 1)
    out_ref[...] = jnp.where(lane == _H, 1.0, p)


def _project_s_body(feat_ref, w_ref, out_ref):
    out_ref[...] = jnp.dot(feat_ref[...], w_ref[...],
                           preferred_element_type=jnp.float32)


def _project(feat, w, body):
    bm = 2000
    return pl.pallas_call(
        body,
        grid=(_N // bm,),
        in_specs=[
            pl.BlockSpec((bm, _D), lambda i: (i, 0)),
            pl.BlockSpec((_D, _L), lambda i: (0, 0)),
        ],
        out_specs=pl.BlockSpec((bm, _L), lambda i: (i, 0)),
        out_shape=jax.ShapeDtypeStruct((_N, _L), jnp.float32),
    )(feat, w)


def _readout_body(parts_ref, w2s_ref, w2n_ref, wfc_ref, out_ref):
    a_vec = parts_ref[0:1, 0:_L]
    b_vec = parts_ref[0:1, _L:2 * _L] + parts_ref[1:2, _L:2 * _L]
    g = (jnp.dot(a_vec, w2s_ref[...], preferred_element_type=jnp.float32)
         + jnp.dot(b_vec, w2n_ref[...], preferred_element_type=jnp.float32))
    g = g * (1.0 / _N)
    o = jnp.dot(g, wfc_ref[...], preferred_element_type=jnp.float32)
    out_ref[...] = jax.nn.sigmoid(o[:, :1])


def _readout(parts, w2s, w2n, wfc):
    return pl.pallas_call(
        _readout_body,
        out_shape=jax.ShapeDtypeStruct((1, 1), jnp.float32),
    )(parts, w2s, w2n, wfc)


_NBUF = 8


def _ring_loop(table_hbm, srcv, dstv, acc_sh, bufs, gsems, ssems):
    def step(i, b, warmup):
        pltpu.make_async_copy(table_hbm.at[srcv.at[i]], bufs[b], gsems[b]).wait()
        pltpu.async_copy(bufs[b], acc_sh.at[dstv.at[i]], ssems[b], add=True)
        j = i + 2
        bj = (b + 2) % _NBUF
        if not warmup:
            pltpu.make_async_copy(bufs[bj], acc_sh.at[dstv.at[j - _NBUF]],
                                  ssems[bj]).wait()
        pltpu.async_copy(table_hbm.at[srcv.at[j]], bufs[bj], gsems[bj])

    def tail_step(i, b):
        pltpu.make_async_copy(table_hbm.at[srcv.at[i]], bufs[b], gsems[b]).wait()
        pltpu.async_copy(bufs[b], acc_sh.at[dstv.at[i]], ssems[b], add=True)

    pltpu.async_copy(table_hbm.at[srcv.at[0]], bufs[0], gsems[0])
    pltpu.async_copy(table_hbm.at[srcv.at[1]], bufs[1], gsems[1])
    for i in range(6):
        step(i, i % _NBUF, True)

    def group(k, carry):
        i0 = _NBUF * k + 6
        for d in range(_NBUF):
            step(i0 + d, (6 + d) % _NBUF, False)
        return carry

    ngroups = (_NCHUNKS - 6 - 7) // _NBUF
    lax.fori_loop(0, ngroups, group, 0)
    for i in range(_NBUF * ngroups + 6, _NCHUNKS):
        b = i % _NBUF
        if i + 2 < _NCHUNKS:
            step(i, b, False)
        else:
            tail_step(i, b)
    for b in range(_NBUF):
        pltpu.make_async_copy(bufs[b], acc_sh.at[dstv.at[0]], ssems[b]).wait()


def _pass1_body(table_hbm, e_hbm, zeros_hbm, out_hbm, srcv, dstv, *scr):
    bufs, acc_sh, gsems, ssems = (scr[:_NBUF], scr[_NBUF],
                                  scr[_NBUF + 1:2 * _NBUF + 1],
                                  scr[2 * _NBUF + 1:])
    cid = lax.axis_index("c")
    sid = lax.axis_index("s")
    rbase = sid * _RPT
    crow = (cid * _NS + sid) * _NCHUNKS
    pltpu.sync_copy(e_hbm.at[0, pl.ds(crow, _NCHUNKS)], srcv)
    pltpu.sync_copy(e_hbm.at[1, pl.ds(crow, _NCHUNKS)], dstv)
    pltpu.sync_copy(zeros_hbm.at[pl.ds(rbase, _RPT)],
                    acc_sh.at[pl.ds(rbase, _RPT)])
    plsc.subcore_barrier()
    _ring_loop(table_hbm, srcv, dstv, acc_sh, bufs, gsems, ssems)
    plsc.subcore_barrier()
    pltpu.sync_copy(acc_sh.at[pl.ds(rbase, _RPT)],
                    out_hbm.at[pl.ds(cid * _NPAD + rbase, _RPT)])


def _mega_body(ps_hbm, acc1_hbm, e_hbm, zeros_hbm, parts_hbm, x1_hbm,
               srcv, dstv, psv, av0, av1, iv, xv, pv, pall, *scr):
    bufs, acc_sh, parts_sh, gsems, ssems = (
        scr[:_NBUF], scr[_NBUF], scr[_NBUF + 1],
        scr[_NBUF + 2:2 * _NBUF + 2], scr[2 * _NBUF + 2:])
    cid = lax.axis_index("c")
    sid = lax.axis_index("s")
    rbase = sid * _RPT
    crow = (cid * _NS + sid) * _NCHUNKS
    pltpu.sync_copy(e_hbm.at[0, pl.ds(crow, _NCHUNKS)], srcv)
    pltpu.sync_copy(e_hbm.at[1, pl.ds(crow, _NCHUNKS)], dstv)
    pltpu.sync_copy(ps_hbm.at[pl.ds(rbase, _RPT)], psv)
    pltpu.sync_copy(acc1_hbm.at[pl.ds(rbase, _RPT)], av0)
    pltpu.sync_copy(acc1_hbm.at[pl.ds(_NPAD + rbase, _RPT)], av1)
    pltpu.sync_copy(zeros_hbm.at[pl.ds(rbase, _RPT)],
                    acc_sh.at[pl.ds(rbase, _RPT)])

    iota16 = lax.broadcasted_iota(jnp.int32, (_L,), 0)

    def _row(ref, r):
        return plsc.load_gather(ref, [jnp.full((_L,), r, jnp.int32), iota16])

    def _setrow(ref, r, x):
        plsc.store_scatter(ref, [jnp.full((_L,), r, jnp.int32), iota16], x)

    off = cid * _NPAD

    def offrow(i, carry):
        ir = jnp.full((_L,), i, jnp.int32)
        for j in range(_CHUNK // _L):
            ic = iota16 + (j * _L)
            plsc.store_scatter(srcv, [ir, ic],
                               plsc.load_gather(srcv, [ir, ic]) + off)
        return carry

    lax.fori_loop(0, _NCHUNKS, offrow, 0)

    mask10 = iota16 < _H

    def xrow(r, apart):
        arow = _row(av0, r) + _row(av1, r)
        degv = jnp.broadcast_to(arow[_H], (_L,))
        invd = 1.0 / jnp.maximum(degv, 1.0)
        x1r = jnp.maximum(_row(psv, r) + arow * invd, 0.0)
        x1r = jnp.where(mask10, x1r, 0.0)
        _setrow(xv, r, x1r)
        _setrow(iv, r, invd)
        return apart + x1r

    apart = lax.fori_loop(0, _RPT, xrow, jnp.zeros((_L,), jnp.float32))
    pltpu.sync_copy(xv, x1_hbm.at[pl.ds(off + rbase, _RPT)])
    plsc.subcore_barrier()

    _ring_loop(x1_hbm, srcv, dstv, acc_sh, bufs, gsems, ssems)
    plsc.subcore_barrier()

    pltpu.sync_copy(acc_sh.at[pl.ds(rbase, _RPT)], av0)

    def brow(r, bpart):
        return bpart + _row(av0, r) * _row(iv, r)

    bpart = lax.fori_loop(0, _RPT, brow, jnp.zeros((_L,), jnp.float32))
    pv[pl.ds(0, _L)] = apart
    pv[pl.ds(_L, _L)] = bpart
    pltpu.sync_copy(pv, parts_sh.at[sid])
    plsc.subcore_barrier()

    @pl.when(sid == 0)
    def _():
        pltpu.sync_copy(parts_sh, pall)

        def red(t, ab):
            tr = jnp.full((_L,), t, jnp.int32)
            pa = plsc.load_gather(pall, [tr, iota16])
            pb = plsc.load_gather(pall, [tr, iota16 + _L])
            return (ab[0] + pa, ab[1] + pb)

        asum, bsum = lax.fori_loop(
            0, _NS, red,
            (jnp.zeros((_L,), jnp.float32), jnp.zeros((_L,), jnp.float32)))
        pv[pl.ds(0, _L)] = asum
        pv[pl.ds(_L, _L)] = bsum
        pltpu.sync_copy(pv, parts_hbm.at[cid])


_SC_PARAMS = pltpu.CompilerParams(use_tc_tiling_on_sc=False,
                                  needs_layout_passes=False)


@functools.cache
def _pass1_call():
    return pl.kernel(
        _pass1_body,
        out_type=jax.ShapeDtypeStruct((_NC * _NPAD, _L), jnp.float32),
        mesh=plsc.VectorSubcoreMesh(core_axis_name="c", subcore_axis_name="s",
                                    num_cores=_NC, num_subcores=_NS),
        scratch_types=[
            pltpu.VMEM((_NCHUNKS, _CHUNK), jnp.int32),
            pltpu.VMEM((_NCHUNKS, _CHUNK), jnp.int32),
        ] + [pltpu.VMEM((_CHUNK, _L), jnp.float32)] * _NBUF + [
            pltpu.VMEM_SHARED((_NPAD, _L), jnp.float32),
        ] + [pltpu.SemaphoreType.DMA] * (2 * _NBUF),
        compiler_params=_SC_PARAMS,
    )


@functools.cache
def _mega_call():
    return pl.kernel(
        _mega_body,
        out_type=[
            jax.ShapeDtypeStruct((_NC, 2 * _L), jnp.float32),
            jax.ShapeDtypeStruct((_NC * _NPAD, _L), jnp.float32),
        ],
        mesh=plsc.VectorSubcoreMesh(core_axis_name="c", subcore_axis_name="s",
                                    num_cores=_NC, num_subcores=_NS),
        scratch_types=[
            pltpu.VMEM((_NCHUNKS, _CHUNK), jnp.int32),
            pltpu.VMEM((_NCHUNKS, _CHUNK), jnp.int32),
            pltpu.VMEM((_RPT, _L), jnp.float32),
            pltpu.VMEM((_RPT, _L), jnp.float32),
            pltpu.VMEM((_RPT, _L), jnp.float32),
            pltpu.VMEM((_RPT, _L), jnp.float32),
            pltpu.VMEM((_RPT, _L), jnp.float32),
            pltpu.VMEM((2 * _L,), jnp.float32),
            pltpu.VMEM((_NS, 2 * _L), jnp.float32),
        ] + [pltpu.VMEM((_CHUNK, _L), jnp.float32)] * _NBUF + [
            pltpu.VMEM_SHARED((_NPAD, _L), jnp.float32),
            pltpu.VMEM_SHARED((_NS, 2 * _L), jnp.float32),
        ] + [pltpu.SemaphoreType.DMA] * (2 * _NBUF),
        compiler_params=_SC_PARAMS,
    )


def _pass1(table, e3, ztbl):
    return _pass1_call()(table, e3, ztbl)


def _mega(ps, acc1, e3, ztbl):
    return _mega_call()(ps, acc1, e3, ztbl)


def kernel(features, edge_index, w1_self, w1_neigh, w2_self, w2_neigh, w_fc1):
    e3 = edge_index.reshape(2, _E // _CHUNK, _CHUNK)
    wn = jnp.pad(w1_neigh, ((0, 0), (0, _L - _H)))
    ws = jnp.pad(w1_self, ((0, 0), (0, _L - _H)))
    w2s = jnp.pad(w2_self, ((0, _L - _H), (0, _L - _H)))
    w2n = jnp.pad(w2_neigh, ((0, _L - _H), (0, _L - _H)))
    wfc = jnp.pad(w_fc1, ((0, _L - _H), (0, _D - 1)))
    ztbl = jnp.zeros((_NPAD, _L), jnp.float32)

    pn = _project(features, wn, _project_n_body)
    acc1 = _pass1(pn, e3, ztbl)
    ps = jnp.pad(_project(features, ws, _project_s_body),
                 ((0, _NPAD - _N), (0, 0)))
    parts, _ = _mega(ps, acc1, e3, ztbl)
    return _readout(parts, w2s, w2n, wfc)


# speedup vs baseline: 1.0293x; 1.0293x over previous
import functools

import jax
import jax.numpy as jnp
from jax import lax
from jax.experimental import pallas as pl
from jax.experimental.pallas import tpu as pltpu
from jax.experimental.pallas import tpu_sc as plsc

_N = 10000
_E = 320000
_D = 128
_H = 10
_L = 16
_NC = 2
_NS = 16
_NW = _NC * _NS
_NPAD = 10240
_RPT = _NPAD // _NS
_EPW = _E // _NW
_CHUNK = 80
_NCHUNKS = _EPW // _CHUNK


def _project_n_body(feat_ref, w_ref, out_ref):
    p = jnp.dot(feat_ref[...], w_ref[...], preferred_element_type=jnp.float32)
    lane = lax.broadcasted_iota(jnp.int32, p.shape, 1)
    out_ref[...] = jnp.where(lane == _H, 1.0, p)


def _project_s_body(feat_ref, w_ref, out_ref):
    out_ref[...] = jnp.dot(feat_ref[...], w_ref[...],
                           preferred_element_type=jnp.float32)


def _project(feat, w, body):
    bm = 2000
    return pl.pallas_call(
        body,
        grid=(_N // bm,),
        in_specs=[
            pl.BlockSpec((bm, _D), lambda i: (i, 0)),
            pl.BlockSpec((_D, _L), lambda i: (0, 0)),
        ],
        out_specs=pl.BlockSpec((bm, _L), lambda i: (i, 0)),
        out_shape=jax.ShapeDtypeStruct((_N, _L), jnp.float32),
    )(feat, w)


def _readout_body(parts_ref, w2s_ref, w2n_ref, wfc_ref, out_ref):
    a_vec = parts_ref[0:1, 0:_L]
    b_vec = parts_ref[0:1, _L:2 * _L] + parts_ref[1:2, _L:2 * _L]
    g = (jnp.dot(a_vec, w2s_ref[...], preferred_element_type=jnp.float32)
         + jnp.dot(b_vec, w2n_ref[...], preferred_element_type=jnp.float32))
    g = g * (1.0 / _N)
    o = jnp.dot(g, wfc_ref[...], preferred_element_type=jnp.float32)
    out_ref[...] = jax.nn.sigmoid(o[:, :1])


def _readout(parts, w2s, w2n, wfc):
    return pl.pallas_call(
        _readout_body,
        out_shape=jax.ShapeDtypeStruct((1, 1), jnp.float32),
    )(parts, w2s, w2n, wfc)


_NBUF = 4


def _ring_loop(table_hbm, srcv, dstv, acc_sh, bufs, gsems, ssems):
    def step(i, b, warmup):
        pltpu.make_async_copy(table_hbm.at[srcv.at[i]], bufs[b], gsems[b]).wait()
        pltpu.async_copy(bufs[b], acc_sh.at[dstv.at[i]], ssems[b], add=True)
        j = i + 2
        bj = (b + 2) % _NBUF
        if not warmup:
            pltpu.make_async_copy(bufs[bj], acc_sh.at[dstv.at[j - _NBUF]],
                                  ssems[bj]).wait()
        pltpu.async_copy(table_hbm.at[srcv.at[j]], bufs[bj], gsems[bj])

    def tail_step(i, b):
        pltpu.make_async_copy(table_hbm.at[srcv.at[i]], bufs[b], gsems[b]).wait()
        pltpu.async_copy(bufs[b], acc_sh.at[dstv.at[i]], ssems[b], add=True)

    pltpu.async_copy(table_hbm.at[srcv.at[0]], bufs[0], gsems[0])
    pltpu.async_copy(table_hbm.at[srcv.at[1]], bufs[1], gsems[1])
    nwarm = _NBUF - 2
    for i in range(nwarm):
        step(i, i % _NBUF, True)

    def group(k, carry):
        i0 = _NBUF * k + nwarm
        for d in range(_NBUF):
            step(i0 + d, (nwarm + d) % _NBUF, False)
        return carry

    ntail = (_NCHUNKS - nwarm) % _NBUF + _NBUF
    ngroups = (_NCHUNKS - nwarm - ntail) // _NBUF
    lax.fori_loop(0, ngroups, group, 0)
    for i in range(_NBUF * ngroups + nwarm, _NCHUNKS):
        b = i % _NBUF
        if i + 2 < _NCHUNKS:
            step(i, b, False)
        else:
            tail_step(i, b)
    for b in range(_NBUF):
        pltpu.make_async_copy(bufs[b], acc_sh.at[dstv.at[0]], ssems[b]).wait()


def _pass1_body(table_hbm, e_hbm, zeros_hbm, out_hbm, srcv, dstv, *scr):
    bufs, acc_sh, gsems, ssems = (scr[:_NBUF], scr[_NBUF],
                                  scr[_NBUF + 1:2 * _NBUF + 1],
                                  scr[2 * _NBUF + 1:])
    cid = lax.axis_index("c")
    sid = lax.axis_index("s")
    rbase = sid * _RPT
    crow = (cid * _NS + sid) * _NCHUNKS
    pltpu.async_copy(e_hbm.at[0, pl.ds(crow, _NCHUNKS)], srcv, gsems[0])
    pltpu.async_copy(e_hbm.at[1, pl.ds(crow, _NCHUNKS)], dstv, gsems[1])
    pltpu.async_copy(zeros_hbm.at[pl.ds(rbase, _RPT)],
                     acc_sh.at[pl.ds(rbase, _RPT)], gsems[2])
    pltpu.make_async_copy(e_hbm.at[0, pl.ds(crow, _NCHUNKS)], srcv,
                          gsems[0]).wait()
    pltpu.make_async_copy(e_hbm.at[1, pl.ds(crow, _NCHUNKS)], dstv,
                          gsems[1]).wait()
    pltpu.make_async_copy(zeros_hbm.at[pl.ds(rbase, _RPT)],
                          acc_sh.at[pl.ds(rbase, _RPT)], gsems[2]).wait()
    plsc.subcore_barrier()
    _ring_loop(table_hbm, srcv, dstv, acc_sh, bufs, gsems, ssems)
    plsc.subcore_barrier()
    pltpu.sync_copy(acc_sh.at[pl.ds(rbase, _RPT)],
                    out_hbm.at[pl.ds(cid * _NPAD + rbase, _RPT)])


def _mega_body(ps_hbm, acc1_hbm, e_hbm, zeros_hbm, parts_hbm, x1_hbm,
               srcv, dstv, psv, av0, av1, iv, xv, pv, pall, *scr):
    bufs, acc_sh, parts_sh, gsems, ssems = (
        scr[:_NBUF], scr[_NBUF], scr[_NBUF + 1],
        scr[_NBUF + 2:2 * _NBUF + 2], scr[2 * _NBUF + 2:])
    cid = lax.axis_index("c")
    sid = lax.axis_index("s")
    rbase = sid * _RPT
    crow = (cid * _NS + sid) * _NCHUNKS
    stages = [
        (e_hbm.at[0, pl.ds(crow, _NCHUNKS)], srcv, gsems[0]),
        (e_hbm.at[1, pl.ds(crow, _NCHUNKS)], dstv, gsems[1]),
        (ps_hbm.at[pl.ds(rbase, _RPT)], psv, gsems[2]),
        (acc1_hbm.at[pl.ds(rbase, _RPT)], av0, gsems[3]),
        (acc1_hbm.at[pl.ds(_NPAD + rbase, _RPT)], av1, ssems[0]),
        (zeros_hbm.at[pl.ds(rbase, _RPT)], acc_sh.at[pl.ds(rbase, _RPT)],
         ssems[1]),
    ]
    for s, t, sem in stages:
        pltpu.async_copy(s, t, sem)
    for s, t, sem in stages:
        pltpu.make_async_copy(s, t, sem).wait()

    iota16 = lax.broadcasted_iota(jnp.int32, (_L,), 0)

    def _row(ref, r):
        return plsc.load_gather(ref, [jnp.full((_L,), r, jnp.int32), iota16])

    def _setrow(ref, r, x):
        plsc.store_scatter(ref, [jnp.full((_L,), r, jnp.int32), iota16], x)

    off = cid * _NPAD

    def offrow(i, carry):
        ir = jnp.full((_L,), i, jnp.int32)
        for j in range(_CHUNK // _L):
            ic = iota16 + (j * _L)
            plsc.store_scatter(srcv, [ir, ic],
                               plsc.load_gather(srcv, [ir, ic]) + off)
        return carry

    lax.fori_loop(0, _NCHUNKS, offrow, 0)

    mask10 = iota16 < _H

    def xrow(r, apart):
        arow = _row(av0, r) + _row(av1, r)
        degv = jnp.broadcast_to(arow[_H], (_L,))
        invd = 1.0 / jnp.maximum(degv, 1.0)
        x1r = jnp.maximum(_row(psv, r) + arow * invd, 0.0)
        x1r = jnp.where(mask10, x1r, 0.0)
        _setrow(xv, r, x1r)
        _setrow(iv, r, invd)
        return apart + x1r

    apart = lax.fori_loop(0, _RPT, xrow, jnp.zeros((_L,), jnp.float32))
    pltpu.sync_copy(xv, x1_hbm.at[pl.ds(off + rbase, _RPT)])
    plsc.subcore_barrier()

    _ring_loop(x1_hbm, srcv, dstv, acc_sh, bufs, gsems, ssems)
    plsc.subcore_barrier()

    pltpu.sync_copy(acc_sh.at[pl.ds(rbase, _RPT)], av0)

    def brow(r, bpart):
        return bpart + _row(av0, r) * _row(iv, r)

    bpart = lax.fori_loop(0, _RPT, brow, jnp.zeros((_L,), jnp.float32))
    pv[pl.ds(0, _L)] = apart
    pv[pl.ds(_L, _L)] = bpart
    pltpu.sync_copy(pv, parts_sh.at[sid])
    plsc.subcore_barrier()

    @pl.when(sid == 0)
    def _():
        pltpu.sync_copy(parts_sh, pall)

        def red(t, ab):
            tr = jnp.full((_L,), t, jnp.int32)
            pa = plsc.load_gather(pall, [tr, iota16])
            pb = plsc.load_gather(pall, [tr, iota16 + _L])
            return (ab[0] + pa, ab[1] + pb)

        asum, bsum = lax.fori_loop(
            0, _NS, red,
            (jnp.zeros((_L,), jnp.float32), jnp.zeros((_L,), jnp.float32)))
        pv[pl.ds(0, _L)] = asum
        pv[pl.ds(_L, _L)] = bsum
        pltpu.sync_copy(pv, parts_hbm.at[cid])


_SC_PARAMS = pltpu.CompilerParams(use_tc_tiling_on_sc=False,
                                  needs_layout_passes=False)


@functools.cache
def _pass1_call():
    return pl.kernel(
        _pass1_body,
        out_type=jax.ShapeDtypeStruct((_NC * _NPAD, _L), jnp.float32),
        mesh=plsc.VectorSubcoreMesh(core_axis_name="c", subcore_axis_name="s",
                                    num_cores=_NC, num_subcores=_NS),
        scratch_types=[
            pltpu.VMEM((_NCHUNKS, _CHUNK), jnp.int32),
            pltpu.VMEM((_NCHUNKS, _CHUNK), jnp.int32),
        ] + [pltpu.VMEM((_CHUNK, _L), jnp.float32)] * _NBUF + [
            pltpu.VMEM_SHARED((_NPAD, _L), jnp.float32),
        ] + [pltpu.SemaphoreType.DMA] * (2 * _NBUF),
        compiler_params=_SC_PARAMS,
    )


@functools.cache
def _mega_call():
    return pl.kernel(
        _mega_body,
        out_type=[
            jax.ShapeDtypeStruct((_NC, 2 * _L), jnp.float32),
            jax.ShapeDtypeStruct((_NC * _NPAD, _L), jnp.float32),
        ],
        mesh=plsc.VectorSubcoreMesh(core_axis_name="c", subcore_axis_name="s",
                                    num_cores=_NC, num_subcores=_NS),
        scratch_types=[
            pltpu.VMEM((_NCHUNKS, _CHUNK), jnp.int32),
            pltpu.VMEM((_NCHUNKS, _CHUNK), jnp.int32),
            pltpu.VMEM((_RPT, _L), jnp.float32),
            pltpu.VMEM((_RPT, _L), jnp.float32),
            pltpu.VMEM((_RPT, _L), jnp.float32),
            pltpu.VMEM((_RPT, _L), jnp.float32),
            pltpu.VMEM((_RPT, _L), jnp.float32),
            pltpu.VMEM((2 * _L,), jnp.float32),
            pltpu.VMEM((_NS, 2 * _L), jnp.float32),
        ] + [pltpu.VMEM((_CHUNK, _L), jnp.float32)] * _NBUF + [
            pltpu.VMEM_SHARED((_NPAD, _L), jnp.float32),
            pltpu.VMEM_SHARED((_NS, 2 * _L), jnp.float32),
        ] + [pltpu.SemaphoreType.DMA] * (2 * _NBUF),
        compiler_params=_SC_PARAMS,
    )


def _pass1(table, e3, ztbl):
    return _pass1_call()(table, e3, ztbl)


def _mega(ps, acc1, e3, ztbl):
    return _mega_call()(ps, acc1, e3, ztbl)


def kernel(features, edge_index, w1_self, w1_neigh, w2_self, w2_neigh, w_fc1):
    e3 = edge_index.reshape(2, _E // _CHUNK, _CHUNK)
    wn = jnp.pad(w1_neigh, ((0, 0), (0, _L - _H)))
    ws = jnp.pad(w1_self, ((0, 0), (0, _L - _H)))
    w2s = jnp.pad(w2_self, ((0, _L - _H), (0, _L - _H)))
    w2n = jnp.pad(w2_neigh, ((0, _L - _H), (0, _L - _H)))
    wfc = jnp.pad(w_fc1, ((0, _L - _H), (0, _D - 1)))
    ztbl = jnp.zeros((_NPAD, _L), jnp.float32)

    pn = _project(features, wn, _project_n_body)
    acc1 = _pass1(pn, e3, ztbl)
    ps = jnp.pad(_project(features, ws, _project_s_body),
                 ((0, _NPAD - _N), (0, 0)))
    parts, _ = _mega(ps, acc1, e3, ztbl)
    return _readout(parts, w2s, w2n, wfc)


# speedup vs baseline: 1.0318x; 1.0025x over previous
import functools

import jax
import jax.numpy as jnp
from jax import lax
from jax.experimental import pallas as pl
from jax.experimental.pallas import tpu as pltpu
from jax.experimental.pallas import tpu_sc as plsc

_N = 10000
_E = 320000
_D = 128
_H = 10
_L = 16
_NC = 2
_NS = 16
_NW = _NC * _NS
_NPAD = 10240
_RPT = _NPAD // _NS
_EPW = _E // _NW
_CHUNK = 80
_NCHUNKS = _EPW // _CHUNK


def _project_n_body(feat_ref, w_ref, out_ref):
    p = jnp.dot(feat_ref[...], w_ref[...], preferred_element_type=jnp.float32)
    lane = lax.broadcasted_iota(jnp.int32, p.shape, 1)
    out_ref[...] = jnp.where(lane == _H, 1.0, p)


def _project_s_body(feat_ref, w_ref, out_ref):
    out_ref[...] = jnp.dot(feat_ref[...], w_ref[...],
                           preferred_element_type=jnp.float32)


def _project(feat, w, body):
    bm = 2000
    return pl.pallas_call(
        body,
        grid=(_N // bm,),
        in_specs=[
            pl.BlockSpec((bm, _D), lambda i: (i, 0)),
            pl.BlockSpec((_D, _L), lambda i: (0, 0)),
        ],
        out_specs=pl.BlockSpec((bm, _L), lambda i: (i, 0)),
        out_shape=jax.ShapeDtypeStruct((_N, _L), jnp.float32),
    )(feat, w)


def _readout_body(parts_ref, w2s_ref, w2n_ref, wfc_ref, out_ref):
    a_vec = parts_ref[0:1, 0:_L]
    b_vec = parts_ref[0:1, _L:2 * _L] + parts_ref[1:2, _L:2 * _L]
    g = (jnp.dot(a_vec, w2s_ref[...], preferred_element_type=jnp.float32)
         + jnp.dot(b_vec, w2n_ref[...], preferred_element_type=jnp.float32))
    g = g * (1.0 / _N)
    o = jnp.dot(g, wfc_ref[...], preferred_element_type=jnp.float32)
    out_ref[...] = jax.nn.sigmoid(o[:, :1])


def _readout(parts, w2s, w2n, wfc):
    return pl.pallas_call(
        _readout_body,
        out_shape=jax.ShapeDtypeStruct((1, 1), jnp.float32),
    )(parts, w2s, w2n, wfc)


_NBUF = 4


def _ring_loop(table_hbm, srcv, dstv, acc_sh, bufs, gsems, ssems):
    def step(i, b, warmup):
        pltpu.make_async_copy(table_hbm.at[srcv.at[i]], bufs[b], gsems[b]).wait()
        pltpu.async_copy(bufs[b], acc_sh.at[dstv.at[i]], ssems[b], add=True)
        j = i + 2
        bj = (b + 2) % _NBUF
        if not warmup:
            pltpu.make_async_copy(bufs[bj], acc_sh.at[dstv.at[j - _NBUF]],
                                  ssems[bj]).wait()
        pltpu.async_copy(table_hbm.at[srcv.at[j]], bufs[bj], gsems[bj])

    def tail_step(i, b):
        pltpu.make_async_copy(table_hbm.at[srcv.at[i]], bufs[b], gsems[b]).wait()
        pltpu.async_copy(bufs[b], acc_sh.at[dstv.at[i]], ssems[b], add=True)

    pltpu.async_copy(table_hbm.at[srcv.at[0]], bufs[0], gsems[0])
    pltpu.async_copy(table_hbm.at[srcv.at[1]], bufs[1], gsems[1])
    nwarm = _NBUF - 2
    for i in range(nwarm):
        step(i, i % _NBUF, True)

    def group(k, carry):
        i0 = _NBUF * k + nwarm
        for d in range(_NBUF):
            step(i0 + d, (nwarm + d) % _NBUF, False)
        return carry

    ntail = (_NCHUNKS - nwarm) % _NBUF + _NBUF
    ngroups = (_NCHUNKS - nwarm - ntail) // _NBUF
    lax.fori_loop(0, ngroups, group, 0)
    for i in range(_NBUF * ngroups + nwarm, _NCHUNKS):
        b = i % _NBUF
        if i + 2 < _NCHUNKS:
            step(i, b, False)
        else:
            tail_step(i, b)
    for b in range(_NBUF):
        pltpu.make_async_copy(bufs[b], acc_sh.at[dstv.at[0]], ssems[b]).wait()


def _pass1_body(table_hbm, e_hbm, zeros_hbm, out_hbm, srcv, dstv, *scr):
    bufs, acc_sh, gsems, ssems = (scr[:_NBUF], scr[_NBUF],
                                  scr[_NBUF + 1:2 * _NBUF + 1],
                                  scr[2 * _NBUF + 1:])
    cid = lax.axis_index("c")
    sid = lax.axis_index("s")
    rbase = sid * _RPT
    crow = (cid * _NS + sid) * _NCHUNKS
    pltpu.async_copy(e_hbm.at[0, pl.ds(crow, _NCHUNKS)], srcv, gsems[0])
    pltpu.async_copy(e_hbm.at[1, pl.ds(crow, _NCHUNKS)], dstv, gsems[1])
    pltpu.async_copy(zeros_hbm.at[pl.ds(rbase, _RPT)],
                     acc_sh.at[pl.ds(rbase, _RPT)], gsems[2])
    pltpu.make_async_copy(e_hbm.at[0, pl.ds(crow, _NCHUNKS)], srcv,
                          gsems[0]).wait()
    pltpu.make_async_copy(e_hbm.at[1, pl.ds(crow, _NCHUNKS)], dstv,
                          gsems[1]).wait()
    pltpu.make_async_copy(zeros_hbm.at[pl.ds(rbase, _RPT)],
                          acc_sh.at[pl.ds(rbase, _RPT)], gsems[2]).wait()
    plsc.subcore_barrier()
    _ring_loop(table_hbm, srcv, dstv, acc_sh, bufs, gsems, ssems)
    plsc.subcore_barrier()
    pltpu.sync_copy(acc_sh.at[pl.ds(rbase, _RPT)],
                    out_hbm.at[pl.ds(cid * _NPAD + rbase, _RPT)])


def _mega_body(ps_hbm, acc1_hbm, e_hbm, zeros_hbm, parts_hbm, x1_hbm,
               srcv, dstv, psv, av0, av1, iv, xv, pv, pall, *scr):
    bufs, acc_sh, parts_sh, gsems, ssems = (
        scr[:_NBUF], scr[_NBUF], scr[_NBUF + 1],
        scr[_NBUF + 2:2 * _NBUF + 2], scr[2 * _NBUF + 2:])
    cid = lax.axis_index("c")
    sid = lax.axis_index("s")
    rbase = sid * _RPT
    crow = (cid * _NS + sid) * _NCHUNKS
    stages = [
        (e_hbm.at[0, pl.ds(crow, _NCHUNKS)], srcv, gsems[0]),
        (e_hbm.at[1, pl.ds(crow, _NCHUNKS)], dstv, gsems[1]),
        (ps_hbm.at[pl.ds(rbase, _RPT)], psv, gsems[2]),
        (acc1_hbm.at[pl.ds(rbase, _RPT)], av0, gsems[3]),
        (acc1_hbm.at[pl.ds(_NPAD + rbase, _RPT)], av1, ssems[0]),
        (zeros_hbm.at[pl.ds(rbase, _RPT)], acc_sh.at[pl.ds(rbase, _RPT)],
         ssems[1]),
    ]
    for s, t, sem in stages:
        pltpu.async_copy(s, t, sem)
    for s, t, sem in stages:
        pltpu.make_async_copy(s, t, sem).wait()

    iota16 = lax.broadcasted_iota(jnp.int32, (_L,), 0)

    off = cid * _NPAD

    def offrow(i, carry):
        ir = jnp.full((_L,), i, jnp.int32)
        for j in range(_CHUNK // _L):
            ic = iota16 + (j * _L)
            plsc.store_scatter(srcv, [ir, ic],
                               plsc.load_gather(srcv, [ir, ic]) + off)
        return carry

    lax.fori_loop(0, _NCHUNKS, offrow, 0)

    mask10 = iota16 < _H

    def xrow(r, apart):
        rv = jnp.full((_L,), r, jnp.int32)
        arow = (plsc.load_gather(av0, [rv, iota16])
                + plsc.load_gather(av1, [rv, iota16]))
        degv = jnp.broadcast_to(arow[_H], (_L,))
        invd = 1.0 / jnp.maximum(degv, 1.0)
        x1r = jnp.maximum(plsc.load_gather(psv, [rv, iota16]) + arow * invd,
                          0.0)
        x1r = jnp.where(mask10, x1r, 0.0)
        plsc.store_scatter(xv, [rv, iota16], x1r)
        plsc.store_scatter(iv, [rv, iota16], invd)
        return apart + x1r

    apart = lax.fori_loop(0, _RPT, xrow, jnp.zeros((_L,), jnp.float32))
    pltpu.sync_copy(xv, x1_hbm.at[pl.ds(off + rbase, _RPT)])
    plsc.subcore_barrier()

    _ring_loop(x1_hbm, srcv, dstv, acc_sh, bufs, gsems, ssems)
    plsc.subcore_barrier()

    pltpu.sync_copy(acc_sh.at[pl.ds(rbase, _RPT)], av0)

    def brow(r, bpart):
        rv = jnp.full((_L,), r, jnp.int32)
        return bpart + (plsc.load_gather(av0, [rv, iota16])
                        * plsc.load_gather(iv, [rv, iota16]))

    bpart = lax.fori_loop(0, _RPT, brow, jnp.zeros((_L,), jnp.float32))
    pv[pl.ds(0, _L)] = apart
    pv[pl.ds(_L, _L)] = bpart
    pltpu.sync_copy(pv, parts_sh.at[sid])
    plsc.subcore_barrier()

    @pl.when(sid == 0)
    def _():
        pltpu.sync_copy(parts_sh, pall)

        def red(t, ab):
            tr = jnp.full((_L,), t, jnp.int32)
            pa = plsc.load_gather(pall, [tr, iota16])
            pb = plsc.load_gather(pall, [tr, iota16 + _L])
            return (ab[0] + pa, ab[1] + pb)

        asum, bsum = lax.fori_loop(
            0, _NS, red,
            (jnp.zeros((_L,), jnp.float32), jnp.zeros((_L,), jnp.float32)))
        pv[pl.ds(0, _L)] = asum
        pv[pl.ds(_L, _L)] = bsum
        pltpu.sync_copy(pv, parts_hbm.at[cid])


_SC_PARAMS = pltpu.CompilerParams(use_tc_tiling_on_sc=False,
                                  needs_layout_passes=False)


@functools.cache
def _pass1_call():
    return pl.kernel(
        _pass1_body,
        out_type=jax.ShapeDtypeStruct((_NC * _NPAD, _L), jnp.float32),
        mesh=plsc.VectorSubcoreMesh(core_axis_name="c", subcore_axis_name="s",
                                    num_cores=_NC, num_subcores=_NS),
        scratch_types=[
            pltpu.VMEM((_NCHUNKS, _CHUNK), jnp.int32),
            pltpu.VMEM((_NCHUNKS, _CHUNK), jnp.int32),
        ] + [pltpu.VMEM((_CHUNK, _L), jnp.float32)] * _NBUF + [
            pltpu.VMEM_SHARED((_NPAD, _L), jnp.float32),
        ] + [pltpu.SemaphoreType.DMA] * (2 * _NBUF),
        compiler_params=_SC_PARAMS,
    )


@functools.cache
def _mega_call():
    return pl.kernel(
        _mega_body,
        out_type=[
            jax.ShapeDtypeStruct((_NC, 2 * _L), jnp.float32),
            jax.ShapeDtypeStruct((_NC * _NPAD, _L), jnp.float32),
        ],
        mesh=plsc.VectorSubcoreMesh(core_axis_name="c", subcore_axis_name="s",
                                    num_cores=_NC, num_subcores=_NS),
        scratch_types=[
            pltpu.VMEM((_NCHUNKS, _CHUNK), jnp.int32),
            pltpu.VMEM((_NCHUNKS, _CHUNK), jnp.int32),
            pltpu.VMEM((_RPT, _L), jnp.float32),
            pltpu.VMEM((_RPT, _L), jnp.float32),
            pltpu.VMEM((_RPT, _L), jnp.float32),
            pltpu.VMEM((_RPT, _L), jnp.float32),
            pltpu.VMEM((_RPT, _L), jnp.float32),
            pltpu.VMEM((2 * _L,), jnp.float32),
            pltpu.VMEM((_NS, 2 * _L), jnp.float32),
        ] + [pltpu.VMEM((_CHUNK, _L), jnp.float32)] * _NBUF + [
            pltpu.VMEM_SHARED((_NPAD, _L), jnp.float32),
            pltpu.VMEM_SHARED((_NS, 2 * _L), jnp.float32),
        ] + [pltpu.SemaphoreType.DMA] * (2 * _NBUF),
        compiler_params=_SC_PARAMS,
    )


def _pass1(table, e3, ztbl):
    return _pass1_call()(table, e3, ztbl)


def _mega(ps, acc1, e3, ztbl):
    return _mega_call()(ps, acc1, e3, ztbl)


def kernel(features, edge_index, w1_self, w1_neigh, w2_self, w2_neigh, w_fc1):
    e3 = edge_index.reshape(2, _E // _CHUNK, _CHUNK)
    wn = jnp.pad(w1_neigh, ((0, 0), (0, _L - _H)))
    ws = jnp.pad(w1_self, ((0, 0), (0, _L - _H)))
    w2s = jnp.pad(w2_self, ((0, _L - _H), (0, _L - _H)))
    w2n = jnp.pad(w2_neigh, ((0, _L - _H), (0, _L - _H)))
    wfc = jnp.pad(w_fc1, ((0, _L - _H), (0, _D - 1)))
    ztbl = jnp.zeros((_NPAD, _L), jnp.float32)

    pn = _project(features, wn, _project_n_body)
    acc1 = _pass1(pn, e3, ztbl)
    ps = jnp.pad(_project(features, ws, _project_s_body),
                 ((0, _NPAD - _N), (0, 0)))
    parts, _ = _mega(ps, acc1, e3, ztbl)
    return _readout(parts, w2s, w2n, wfc)
